# baseline (speedup 1.0000x reference)
import functools

import jax
import jax.numpy as jnp
from jax import lax
from jax.experimental import pallas as pl
from jax.experimental.pallas import tpu as pltpu
from jax.experimental.pallas import tpu_sc as plsc

_N = 10000
_NPAD = 10240
_VOCAB = 1001
_VPAD = 1024
_HEADS = 2
_HID = 128
_GRAPHS = 64
_ETOT = 320000 + _N
_CHUNK = 128
_TILES = 16
_CPT = 168
_G8 = _CPT // 8
_EPAD = _CPT * _CHUNK * _TILES
_NPT = _NPAD // _TILES
_NCH = _NPT // _CHUNK


def _prep_body(emb_ref, w_ref, as_ref, ad_ref, t2_ref, asv_ref, adv_ref):
    hh = jnp.dot(emb_ref[...], w_ref[...], preferred_element_type=jnp.float32)
    h0 = hh[:, :_HID]
    h1 = hh[:, _HID:]
    t2_ref[0] = h0
    t2_ref[1] = h1
    s0 = jnp.sum(h0 * as_ref[0][None, :], axis=1)
    s1 = jnp.sum(h1 * as_ref[1][None, :], axis=1)
    asv_ref[...] = jnp.concatenate([s0[None], s1[None]], axis=0)
    d0 = jnp.sum(h0 * ad_ref[0][None, :], axis=1)
    d1 = jnp.sum(h1 * ad_ref[1][None, :], axis=1)
    adv_ref[...] = jnp.concatenate([d0[None], d1[None]], axis=0)


def _prep(emb_pad, W1, att_s, att_d):
    return pl.pallas_call(
        _prep_body,
        out_shape=[
            jax.ShapeDtypeStruct((_HEADS, _VPAD, _HID), jnp.float32),
            jax.ShapeDtypeStruct((_HEADS, _VPAD), jnp.float32),
            jax.ShapeDtypeStruct((_HEADS, _VPAD), jnp.float32),
        ],
    )(emb_pad, W1, att_s, att_d)


def _node_prep_body(xv_hbm, t2_hbm, asv_hbm, adv_hbm, hn_hbm, asn_hbm,
                    adn_hbm, idxb, offb, rows, sb, sem):
    cid = lax.axis_index("c")
    sid = lax.axis_index("s")
    vbase = cid * _VPAD

    def chunk(j, carry):
        nb = sid * _NPT + j * _CHUNK
        pltpu.sync_copy(xv_hbm.at[pl.ds(nb, _CHUNK)], idxb)
        for k in range(_CHUNK // 16):
            sl = pl.ds(k * 16, 16)
            offb[sl] = idxb[sl] + vbase
        ob = cid * _NPAD + nb
        pltpu.async_copy(t2_hbm.at[offb], rows, sem).wait()
        pltpu.sync_copy(rows, hn_hbm.at[pl.ds(ob, _CHUNK)])
        pltpu.async_copy(asv_hbm.at[offb], sb, sem).wait()
        pltpu.sync_copy(sb, asn_hbm.at[pl.ds(ob, _CHUNK)])
        pltpu.async_copy(adv_hbm.at[offb], sb, sem).wait()
        pltpu.sync_copy(sb, adn_hbm.at[pl.ds(ob, _CHUNK)])
        return carry

    lax.fori_loop(0, _NCH, chunk, 0)


def _edges_body(hn_hbm, asn_hbm, adn_hbm, src_hbm, dst_hbm, z_hbm,
                num_hbm, den_hbm,
                src8, dst8, offs8, offd8, asb8, adb8, eeb8, rows_a, rows_b,
                num_sh, den_sh, sem_s, sem_g, sem_n):
    cid = lax.axis_index("c")
    sid = lax.axis_index("s")
    hbase = cid * _NPAD
    rbufs = (rows_a, rows_b)

    pltpu.sync_copy(z_hbm, rows_a)

    def zero_chunk(j, carry):
        nb = sid * _NPT + j * _CHUNK
        pltpu.sync_copy(rows_a, num_sh.at[pl.ds(nb, _CHUNK)])
        pltpu.sync_copy(rows_a.at[0], den_sh.at[pl.ds(nb, _CHUNK)])
        return carry

    lax.fori_loop(0, _NCH, zero_chunk, 0)
    plsc.subcore_barrier()

    lanes = lax.iota(jnp.int32, 16)

    def edge_group(g, carry):
        pltpu.sync_copy(src_hbm.at[sid, pl.ds(g * 8, 8)], src8)
        pltpu.sync_copy(dst_hbm.at[sid, pl.ds(g * 8, 8)], dst8)
        for c in range(8):
            for k in range(_CHUNK // 16):
                sl = pl.ds(k * 16, 16)
                offs8[c, sl] = src8[c, sl] + hbase
                offd8[c, sl] = dst8[c, sl] + hbase
        descs = []
        for c in range(8):
            descs.append(pltpu.async_copy(asn_hbm.at[offs8.at[c]],
                                          asb8.at[c], sem_s))
            descs.append(pltpu.async_copy(adn_hbm.at[offd8.at[c]],
                                          adb8.at[c], sem_s))
        for d in descs:
            d.wait()
        ebase0 = (sid * _CPT + g * 8) * _CHUNK
        for c in range(8):
            for k in range(_CHUNK // 16):
                sl = pl.ds(k * 16, 16)
                s = asb8[c, sl] + adb8[c, sl]
                lr = jnp.maximum(s, s * 0.2)
                valid = (ebase0 + c * _CHUNK + k * 16 + lanes) < _ETOT
                eeb8[c, sl] = jnp.exp(lr) * jnp.where(valid, 1.0, 0.0)
        dd = [pltpu.async_copy(eeb8.at[c], den_sh.at[dst8.at[c]], sem_s,
                               add=True) for c in range(8)]
        for d in dd:
            d.wait()

        pending = pltpu.async_copy(hn_hbm.at[offs8.at[0]], rows_a, sem_g)
        scat = [None, None]
        for c in range(8):
            pending.wait()
            if c < 7:
                if scat[(c + 1) % 2] is not None:
                    scat[(c + 1) % 2].wait()
                    scat[(c + 1) % 2] = None
                pending = pltpu.async_copy(hn_hbm.at[offs8.at[c + 1]],
                                           rbufs[(c + 1) % 2], sem_g)
            rows = rbufs[c % 2]

            def rowmul(q, rcarry):
                ee16 = eeb8[c, pl.ds(q * 16, 16)]
                for j in range(16):
                    b = jnp.full((16,), ee16[j], jnp.float32)
                    e = q * 16 + j
                    for k in range(_HID // 16):
                        sl = pl.ds(k * 16, 16)
                        rows[e, sl] = rows[e, sl] * b
                return rcarry

            lax.fori_loop(0, _CHUNK // 16, rowmul, 0)
            scat[c % 2] = pltpu.async_copy(rows, num_sh.at[dst8.at[c]],
                                           sem_n, add=True)
        for s in scat:
            if s is not None:
                s.wait()
        return carry

    lax.fori_loop(0, _G8, edge_group, 0)
    plsc.subcore_barrier()

    def writeback(j, carry):
        nb = sid * _NPT + j * _CHUNK
        pltpu.sync_copy(num_sh.at[pl.ds(nb, _CHUNK)],
                        num_hbm.at[pl.ds(hbase + nb, _CHUNK)])
        pltpu.sync_copy(den_sh.at[pl.ds(nb, _CHUNK)],
                        den_hbm.at[pl.ds(hbase + nb, _CHUNK)])
        return carry

    lax.fori_loop(0, _NCH, writeback, 0)


@functools.cache
def _sc_kernels():
    mesh = plsc.VectorSubcoreMesh(core_axis_name="c", subcore_axis_name="s")
    node_prep = pl.kernel(
        _node_prep_body,
        out_type=[
            jax.ShapeDtypeStruct((_HEADS * _NPAD, _HID), jnp.float32),
            jax.ShapeDtypeStruct((_HEADS * _NPAD,), jnp.float32),
            jax.ShapeDtypeStruct((_HEADS * _NPAD,), jnp.float32),
        ],
        mesh=mesh,
        scratch_types=[
            pltpu.VMEM((_CHUNK,), jnp.int32),
            pltpu.VMEM((_CHUNK,), jnp.int32),
            pltpu.VMEM((_CHUNK, _HID), jnp.float32),
            pltpu.VMEM((_CHUNK,), jnp.float32),
            pltpu.SemaphoreType.DMA,
        ],
    )
    edges = pl.kernel(
        _edges_body,
        out_type=[
            jax.ShapeDtypeStruct((_HEADS * _NPAD, _HID), jnp.float32),
            jax.ShapeDtypeStruct((_HEADS * _NPAD,), jnp.float32),
        ],
        mesh=mesh,
        scratch_types=[
            pltpu.VMEM((8, _CHUNK), jnp.int32),
            pltpu.VMEM((8, _CHUNK), jnp.int32),
            pltpu.VMEM((8, _CHUNK), jnp.int32),
            pltpu.VMEM((8, _CHUNK), jnp.int32),
            pltpu.VMEM((8, _CHUNK), jnp.float32),
            pltpu.VMEM((8, _CHUNK), jnp.float32),
            pltpu.VMEM((8, _CHUNK), jnp.float32),
            pltpu.VMEM((_CHUNK, _HID), jnp.float32),
            pltpu.VMEM((_CHUNK, _HID), jnp.float32),
            pltpu.VMEM_SHARED((_NPAD, _HID), jnp.float32),
            pltpu.VMEM_SHARED((_NPAD,), jnp.float32),
            pltpu.SemaphoreType.DMA,
            pltpu.SemaphoreType.DMA,
            pltpu.SemaphoreType.DMA,
        ],
    )
    return node_prep, edges


def _mid_body(num_ref, den_ref, b1_ref, a1_ref, w2_ref, as2_ref, ad2_ref,
              h2_ref, asn2_ref, adn2_ref):
    o0 = num_ref[0] / (den_ref[0][:, None] + 1e-16) + b1_ref[0][None, :]
    o1 = num_ref[1] / (den_ref[1][:, None] + 1e-16) + b1_ref[1][None, :]
    x = jnp.concatenate([o0, o1], axis=1)
    a1 = a1_ref[0, 0]
    x = jnp.where(x >= 0, x, a1 * x)
    h2 = jnp.dot(x, w2_ref[...], preferred_element_type=jnp.float32)
    g0 = h2[:, :_HID]
    g1 = h2[:, _HID:]
    h2_ref[0] = g0
    h2_ref[1] = g1
    s0 = jnp.sum(g0 * as2_ref[0][None, :], axis=1)
    s1 = jnp.sum(g1 * as2_ref[1][None, :], axis=1)
    asn2_ref[...] = jnp.concatenate([s0[None], s1[None]], axis=0)
    d0 = jnp.sum(g0 * ad2_ref[0][None, :], axis=1)
    d1 = jnp.sum(g1 * ad2_ref[1][None, :], axis=1)
    adn2_ref[...] = jnp.concatenate([d0[None], d1[None]], axis=0)


_MID_R = 256


def _mid(num1, den1, b1r, a1r, W2, att_s2, att_d2):
    grid = (_NPAD // _MID_R,)
    return pl.pallas_call(
        _mid_body,
        grid=grid,
        in_specs=[
            pl.BlockSpec((_HEADS, _MID_R, _HID), lambda i: (0, i, 0)),
            pl.BlockSpec((_HEADS, _MID_R), lambda i: (0, i)),
            pl.BlockSpec((_HEADS, _HID), lambda i: (0, 0)),
            pl.BlockSpec((1, 1), lambda i: (0, 0)),
            pl.BlockSpec((2 * _HID, 2 * _HID), lambda i: (0, 0)),
            pl.BlockSpec((_HEADS, _HID), lambda i: (0, 0)),
            pl.BlockSpec((_HEADS, _HID), lambda i: (0, 0)),
        ],
        out_specs=[
            pl.BlockSpec((_HEADS, _MID_R, _HID), lambda i: (0, i, 0)),
            pl.BlockSpec((_HEADS, _MID_R), lambda i: (0, i)),
            pl.BlockSpec((_HEADS, _MID_R), lambda i: (0, i)),
        ],
        out_shape=[
            jax.ShapeDtypeStruct((_HEADS, _NPAD, _HID), jnp.float32),
            jax.ShapeDtypeStruct((_HEADS, _NPAD), jnp.float32),
            jax.ShapeDtypeStruct((_HEADS, _NPAD), jnp.float32),
        ],
    )(num1, den1, b1r, a1r, W2, att_s2, att_d2)


_FIN_R = 128


def _final_body(num_ref, den_ref, b2_ref, gw1_ref, gb1_ref, ag_ref, gw2_ref,
                gb2_ref, batch_ref, oc_ref, hid_ref, hid_acc, gden_acc):
    i = pl.program_id(0)
    o0 = num_ref[0] / (den_ref[0][:, None] + 1e-16) + b2_ref[0][None, :]
    o1 = num_ref[1] / (den_ref[1][:, None] + 1e-16) + b2_ref[1][None, :]
    oc = jnp.concatenate([o0, o1], axis=1)
    oc_ref[...] = oc
    g = jnp.dot(oc, gw1_ref[...], preferred_element_type=jnp.float32)
    g = g + gb1_ref[...]
    ag = ag_ref[0, 0]
    g = jnp.where(g >= 0, g, ag * g)
    gate = jnp.sum(g * gw2_ref[...], axis=1, keepdims=True) + gb2_ref[0, 0]
    ge = jnp.exp(gate)
    onehot = jnp.equal(
        lax.broadcasted_iota(jnp.int32, (_GRAPHS, _FIN_R), 0),
        batch_ref[0, 0][None, :]).astype(jnp.float32)

    @pl.when(i == 0)
    def _():
        hid_acc[...] = jnp.zeros_like(hid_acc)
        gden_acc[...] = jnp.zeros_like(gden_acc)

    hid_acc[...] += jnp.dot(onehot, oc * ge,
                            preferred_element_type=jnp.float32)
    gden_acc[...] += jnp.dot(onehot, jnp.broadcast_to(ge, (_FIN_R, _HID)),
                             preferred_element_type=jnp.float32)

    @pl.when(i == pl.num_programs(0) - 1)
    def _():
        hid_ref[...] = hid_acc[...] / (gden_acc[:, 0:1] + 1e-16)


def _final(num2, den2, b2r, gw1, gb1r, agr, gw2r, gb2r, batch2d):
    grid = (_NPAD // _FIN_R,)
    return pl.pallas_call(
        _final_body,
        grid=grid,
        in_specs=[
            pl.BlockSpec((_HEADS, _FIN_R, _HID), lambda i: (0, i, 0)),
            pl.BlockSpec((_HEADS, _FIN_R), lambda i: (0, i)),
            pl.BlockSpec((_HEADS, _HID), lambda i: (0, 0)),
            pl.BlockSpec((2 * _HID, _HID), lambda i: (0, 0)),
            pl.BlockSpec((1, _HID), lambda i: (0, 0)),
            pl.BlockSpec((1, 1), lambda i: (0, 0)),
            pl.BlockSpec((1, _HID), lambda i: (0, 0)),
            pl.BlockSpec((1, 1), lambda i: (0, 0)),
            pl.BlockSpec((1, 1, _FIN_R), lambda i: (i, 0, 0)),
        ],
        out_specs=[
            pl.BlockSpec((_FIN_R, 2 * _HID), lambda i: (i, 0)),
            pl.BlockSpec((_GRAPHS, 2 * _HID), lambda i: (0, 0)),
        ],
        out_shape=[
            jax.ShapeDtypeStruct((_NPAD, 2 * _HID), jnp.float32),
            jax.ShapeDtypeStruct((_GRAPHS, 2 * _HID), jnp.float32),
        ],
        scratch_shapes=[
            pltpu.VMEM((_GRAPHS, 2 * _HID), jnp.float32),
            pltpu.VMEM((_GRAPHS, _HID), jnp.float32),
        ],
    )(num2, den2, b2r, gw1, gb1r, agr, gw2r, gb2r, batch2d)


def kernel(x, edge_index, batch_idx, emb_table, W1, att_src1, att_dst1, b1,
           a1, W2, att_src2, att_dst2, b2, gw1, gb1, ag, gw2, gb2):
    f32 = jnp.float32
    xv = jnp.pad(x[:, 0], (0, _NPAD - _N))
    emb_pad = jnp.pad(emb_table, ((0, _VPAD - _VOCAB), (0, 0)))

    loop = jnp.arange(_N, dtype=edge_index.dtype)
    src = jnp.concatenate([edge_index[0], loop])
    dst = jnp.concatenate([edge_index[1], loop])
    npad = _EPAD - _ETOT
    src2d = jnp.pad(src, (0, npad)).reshape(_TILES, _CPT, _CHUNK)
    dst2d = jnp.pad(dst, (0, npad)).reshape(_TILES, _CPT, _CHUNK)
    z = jnp.zeros((_CHUNK, _HID), f32)
    batch2d = jnp.pad(batch_idx.astype(jnp.int32), (0, _NPAD - _N),
                      constant_values=_GRAPHS).reshape(
                          _NPAD // _FIN_R, 1, _FIN_R)

    _node_prep, _edges = _sc_kernels()
    t2, asv, adv = _prep(emb_pad, W1, att_src1, att_dst1)
    t2f = t2.reshape(_HEADS * _VPAD, _HID)
    hn1, asn1, adn1 = _node_prep(xv, t2f, asv.reshape(-1), adv.reshape(-1))
    num1, den1 = _edges(hn1, asn1, adn1, src2d, dst2d, z)

    h2, asn2, adn2 = _mid(num1.reshape(_HEADS, _NPAD, _HID),
                          den1.reshape(_HEADS, _NPAD),
                          b1.reshape(_HEADS, _HID),
                          jnp.asarray(a1, f32).reshape(1, 1),
                          W2, att_src2, att_dst2)
    num2, den2 = _edges(h2.reshape(_HEADS * _NPAD, _HID),
                        asn2.reshape(-1), adn2.reshape(-1),
                        src2d, dst2d, z)

    oc_pad, hidden = _final(num2.reshape(_HEADS, _NPAD, _HID),
                            den2.reshape(_HEADS, _NPAD),
                            b2.reshape(_HEADS, _HID),
                            gw1, gb1.reshape(1, _HID),
                            jnp.asarray(ag, f32).reshape(1, 1),
                            gw2.reshape(1, _HID),
                            jnp.asarray(gb2, f32).reshape(1, 1),
                            batch2d)
    return oc_pad[:_N], hidden


# speedup vs baseline: 1.0149x; 1.0149x over previous
import functools

import jax
import jax.numpy as jnp
from jax import lax
from jax.experimental import pallas as pl
from jax.experimental.pallas import tpu as pltpu
from jax.experimental.pallas import tpu_sc as plsc

_N = 10000
_NPAD = 10240
_VOCAB = 1001
_VPAD = 1024
_HEADS = 2
_HID = 128
_GRAPHS = 64
_ETOT = 320000 + _N
_CHUNK = 128
_TILES = 16
_CPT = 168
_G8 = _CPT // 8
_EPAD = _CPT * _CHUNK * _TILES
_NPT = _NPAD // _TILES
_NCH = _NPT // _CHUNK


def _prep_body(emb_ref, w_ref, as_ref, ad_ref, t2_ref, asv_ref, adv_ref):
    hh = jnp.dot(emb_ref[...], w_ref[...], preferred_element_type=jnp.float32)
    h0 = hh[:, :_HID]
    h1 = hh[:, _HID:]
    t2_ref[0] = h0
    t2_ref[1] = h1
    s0 = jnp.sum(h0 * as_ref[0][None, :], axis=1)
    s1 = jnp.sum(h1 * as_ref[1][None, :], axis=1)
    asv_ref[...] = jnp.concatenate([s0[None], s1[None]], axis=0)
    d0 = jnp.sum(h0 * ad_ref[0][None, :], axis=1)
    d1 = jnp.sum(h1 * ad_ref[1][None, :], axis=1)
    adv_ref[...] = jnp.concatenate([d0[None], d1[None]], axis=0)


def _prep(emb_pad, W1, att_s, att_d):
    return pl.pallas_call(
        _prep_body,
        out_shape=[
            jax.ShapeDtypeStruct((_HEADS, _VPAD, _HID), jnp.float32),
            jax.ShapeDtypeStruct((_HEADS, _VPAD), jnp.float32),
            jax.ShapeDtypeStruct((_HEADS, _VPAD), jnp.float32),
        ],
    )(emb_pad, W1, att_s, att_d)


def _node_prep_body(xv_hbm, t2_hbm, asv_hbm, adv_hbm, hn_hbm, asn_hbm,
                    adn_hbm, idxb, offb, rows, sb, sem):
    cid = lax.axis_index("c")
    sid = lax.axis_index("s")
    vbase = cid * _VPAD

    def chunk(j, carry):
        nb = sid * _NPT + j * _CHUNK
        pltpu.sync_copy(xv_hbm.at[pl.ds(nb, _CHUNK)], idxb)
        for k in range(_CHUNK // 16):
            sl = pl.ds(k * 16, 16)
            offb[sl] = idxb[sl] + vbase
        ob = cid * _NPAD + nb
        pltpu.async_copy(t2_hbm.at[offb], rows, sem).wait()
        pltpu.sync_copy(rows, hn_hbm.at[pl.ds(ob, _CHUNK)])
        pltpu.async_copy(asv_hbm.at[offb], sb, sem).wait()
        pltpu.sync_copy(sb, asn_hbm.at[pl.ds(ob, _CHUNK)])
        pltpu.async_copy(adv_hbm.at[offb], sb, sem).wait()
        pltpu.sync_copy(sb, adn_hbm.at[pl.ds(ob, _CHUNK)])
        return carry

    lax.fori_loop(0, _NCH, chunk, 0)


def _edges_body(hn_hbm, asn_hbm, adn_hbm, src_hbm, dst_hbm, z_hbm,
                num_hbm, den_hbm,
                src8, dst8, offs8, offd8, asb8, adb8, eeb8, rows_a, rows_b,
                num_sh, den_sh, sem_s, sem_g, sem_n):
    cid = lax.axis_index("c")
    sid = lax.axis_index("s")
    hbase = cid * _NPAD
    rbufs = (rows_a, rows_b)

    pltpu.sync_copy(z_hbm, rows_a)

    def zero_chunk(j, carry):
        nb = sid * _NPT + j * _CHUNK
        pltpu.sync_copy(rows_a, num_sh.at[pl.ds(nb, _CHUNK)])
        pltpu.sync_copy(rows_a.at[0], den_sh.at[pl.ds(nb, _CHUNK)])
        return carry

    lax.fori_loop(0, _NCH, zero_chunk, 0)
    plsc.subcore_barrier()

    lanes = lax.iota(jnp.int32, 16)

    def edge_group(g, carry):
        pltpu.sync_copy(src_hbm.at[sid, pl.ds(g * 8, 8)], src8)
        pltpu.sync_copy(dst_hbm.at[sid, pl.ds(g * 8, 8)], dst8)
        for c in range(8):
            for k in range(_CHUNK // 16):
                sl = pl.ds(k * 16, 16)
                offs8[c, sl] = src8[c, sl] + hbase
                offd8[c, sl] = dst8[c, sl] + hbase
        descs = []
        for c in range(8):
            descs.append(pltpu.async_copy(asn_hbm.at[offs8.at[c]],
                                          asb8.at[c], sem_s))
            descs.append(pltpu.async_copy(adn_hbm.at[offd8.at[c]],
                                          adb8.at[c], sem_s))
        for d in descs:
            d.wait()
        ebase0 = (sid * _CPT + g * 8) * _CHUNK
        for c in range(8):
            for k in range(_CHUNK // 16):
                sl = pl.ds(k * 16, 16)
                s = asb8[c, sl] + adb8[c, sl]
                lr = jnp.maximum(s, s * 0.2)
                valid = (ebase0 + c * _CHUNK + k * 16 + lanes) < _ETOT
                eeb8[c, sl] = jnp.exp(lr) * jnp.where(valid, 1.0, 0.0)
        dd = [pltpu.async_copy(eeb8.at[c], den_sh.at[dst8.at[c]], sem_s,
                               add=True) for c in range(8)]

        pending = pltpu.async_copy(hn_hbm.at[offs8.at[0]], rows_a, sem_g)
        scat = [None, None]
        for c in range(8):
            pending.wait()
            if c < 7:
                if scat[(c + 1) % 2] is not None:
                    scat[(c + 1) % 2].wait()
                    scat[(c + 1) % 2] = None
                pending = pltpu.async_copy(hn_hbm.at[offs8.at[c + 1]],
                                           rbufs[(c + 1) % 2], sem_g)
            rows = rbufs[c % 2]

            def rowmul(q, rcarry):
                ee16 = eeb8[c, pl.ds(q * 16, 16)]
                for j in range(16):
                    b = jnp.full((16,), ee16[j], jnp.float32)
                    e = q * 16 + j
                    for k in range(_HID // 16):
                        sl = pl.ds(k * 16, 16)
                        rows[e, sl] = rows[e, sl] * b
                return rcarry

            lax.fori_loop(0, _CHUNK // 16, rowmul, 0)
            scat[c % 2] = pltpu.async_copy(rows, num_sh.at[dst8.at[c]],
                                           sem_n, add=True)
        for d in dd:
            d.wait()
        for s in scat:
            if s is not None:
                s.wait()
        return carry

    lax.fori_loop(0, _G8, edge_group, 0)
    plsc.subcore_barrier()

    def writeback(j, carry):
        nb = sid * _NPT + j * _CHUNK
        pltpu.sync_copy(num_sh.at[pl.ds(nb, _CHUNK)],
                        num_hbm.at[pl.ds(hbase + nb, _CHUNK)])
        pltpu.sync_copy(den_sh.at[pl.ds(nb, _CHUNK)],
                        den_hbm.at[pl.ds(hbase + nb, _CHUNK)])
        return carry

    lax.fori_loop(0, _NCH, writeback, 0)


@functools.cache
def _sc_kernels():
    mesh = plsc.VectorSubcoreMesh(core_axis_name="c", subcore_axis_name="s")
    node_prep = pl.kernel(
        _node_prep_body,
        out_type=[
            jax.ShapeDtypeStruct((_HEADS * _NPAD, _HID), jnp.float32),
            jax.ShapeDtypeStruct((_HEADS * _NPAD,), jnp.float32),
            jax.ShapeDtypeStruct((_HEADS * _NPAD,), jnp.float32),
        ],
        mesh=mesh,
        scratch_types=[
            pltpu.VMEM((_CHUNK,), jnp.int32),
            pltpu.VMEM((_CHUNK,), jnp.int32),
            pltpu.VMEM((_CHUNK, _HID), jnp.float32),
            pltpu.VMEM((_CHUNK,), jnp.float32),
            pltpu.SemaphoreType.DMA,
        ],
    )
    edges = pl.kernel(
        _edges_body,
        out_type=[
            jax.ShapeDtypeStruct((_HEADS * _NPAD, _HID), jnp.float32),
            jax.ShapeDtypeStruct((_HEADS * _NPAD,), jnp.float32),
        ],
        mesh=mesh,
        scratch_types=[
            pltpu.VMEM((8, _CHUNK), jnp.int32),
            pltpu.VMEM((8, _CHUNK), jnp.int32),
            pltpu.VMEM((8, _CHUNK), jnp.int32),
            pltpu.VMEM((8, _CHUNK), jnp.int32),
            pltpu.VMEM((8, _CHUNK), jnp.float32),
            pltpu.VMEM((8, _CHUNK), jnp.float32),
            pltpu.VMEM((8, _CHUNK), jnp.float32),
            pltpu.VMEM((_CHUNK, _HID), jnp.float32),
            pltpu.VMEM((_CHUNK, _HID), jnp.float32),
            pltpu.VMEM_SHARED((_NPAD, _HID), jnp.float32),
            pltpu.VMEM_SHARED((_NPAD,), jnp.float32),
            pltpu.SemaphoreType.DMA,
            pltpu.SemaphoreType.DMA,
            pltpu.SemaphoreType.DMA,
        ],
    )
    return node_prep, edges


def _mid_body(num_ref, den_ref, b1_ref, a1_ref, w2_ref, as2_ref, ad2_ref,
              h2_ref, asn2_ref, adn2_ref):
    o0 = num_ref[0] / (den_ref[0][:, None] + 1e-16) + b1_ref[0][None, :]
    o1 = num_ref[1] / (den_ref[1][:, None] + 1e-16) + b1_ref[1][None, :]
    x = jnp.concatenate([o0, o1], axis=1)
    a1 = a1_ref[0, 0]
    x = jnp.where(x >= 0, x, a1 * x)
    h2 = jnp.dot(x, w2_ref[...], preferred_element_type=jnp.float32)
    g0 = h2[:, :_HID]
    g1 = h2[:, _HID:]
    h2_ref[0] = g0
    h2_ref[1] = g1
    s0 = jnp.sum(g0 * as2_ref[0][None, :], axis=1)
    s1 = jnp.sum(g1 * as2_ref[1][None, :], axis=1)
    asn2_ref[...] = jnp.concatenate([s0[None], s1[None]], axis=0)
    d0 = jnp.sum(g0 * ad2_ref[0][None, :], axis=1)
    d1 = jnp.sum(g1 * ad2_ref[1][None, :], axis=1)
    adn2_ref[...] = jnp.concatenate([d0[None], d1[None]], axis=0)


_MID_R = 256


def _mid(num1, den1, b1r, a1r, W2, att_s2, att_d2):
    grid = (_NPAD // _MID_R,)
    return pl.pallas_call(
        _mid_body,
        grid=grid,
        in_specs=[
            pl.BlockSpec((_HEADS, _MID_R, _HID), lambda i: (0, i, 0)),
            pl.BlockSpec((_HEADS, _MID_R), lambda i: (0, i)),
            pl.BlockSpec((_HEADS, _HID), lambda i: (0, 0)),
            pl.BlockSpec((1, 1), lambda i: (0, 0)),
            pl.BlockSpec((2 * _HID, 2 * _HID), lambda i: (0, 0)),
            pl.BlockSpec((_HEADS, _HID), lambda i: (0, 0)),
            pl.BlockSpec((_HEADS, _HID), lambda i: (0, 0)),
        ],
        out_specs=[
            pl.BlockSpec((_HEADS, _MID_R, _HID), lambda i: (0, i, 0)),
            pl.BlockSpec((_HEADS, _MID_R), lambda i: (0, i)),
            pl.BlockSpec((_HEADS, _MID_R), lambda i: (0, i)),
        ],
        out_shape=[
            jax.ShapeDtypeStruct((_HEADS, _NPAD, _HID), jnp.float32),
            jax.ShapeDtypeStruct((_HEADS, _NPAD), jnp.float32),
            jax.ShapeDtypeStruct((_HEADS, _NPAD), jnp.float32),
        ],
    )(num1, den1, b1r, a1r, W2, att_s2, att_d2)


_FIN_R = 128


def _final_body(num_ref, den_ref, b2_ref, gw1_ref, gb1_ref, ag_ref, gw2_ref,
                gb2_ref, batch_ref, oc_ref, hid_ref, hid_acc, gden_acc):
    i = pl.program_id(0)
    o0 = num_ref[0] / (den_ref[0][:, None] + 1e-16) + b2_ref[0][None, :]
    o1 = num_ref[1] / (den_ref[1][:, None] + 1e-16) + b2_ref[1][None, :]
    oc = jnp.concatenate([o0, o1], axis=1)
    oc_ref[...] = oc
    g = jnp.dot(oc, gw1_ref[...], preferred_element_type=jnp.float32)
    g = g + gb1_ref[...]
    ag = ag_ref[0, 0]
    g = jnp.where(g >= 0, g, ag * g)
    gate = jnp.sum(g * gw2_ref[...], axis=1, keepdims=True) + gb2_ref[0, 0]
    ge = jnp.exp(gate)
    onehot = jnp.equal(
        lax.broadcasted_iota(jnp.int32, (_GRAPHS, _FIN_R), 0),
        batch_ref[0, 0][None, :]).astype(jnp.float32)

    @pl.when(i == 0)
    def _():
        hid_acc[...] = jnp.zeros_like(hid_acc)
        gden_acc[...] = jnp.zeros_like(gden_acc)

    hid_acc[...] += jnp.dot(onehot, oc * ge,
                            preferred_element_type=jnp.float32)
    gden_acc[...] += jnp.dot(onehot, jnp.broadcast_to(ge, (_FIN_R, _HID)),
                             preferred_element_type=jnp.float32)

    @pl.when(i == pl.num_programs(0) - 1)
    def _():
        hid_ref[...] = hid_acc[...] / (gden_acc[:, 0:1] + 1e-16)


def _final(num2, den2, b2r, gw1, gb1r, agr, gw2r, gb2r, batch2d):
    grid = (_NPAD // _FIN_R,)
    return pl.pallas_call(
        _final_body,
        grid=grid,
        in_specs=[
            pl.BlockSpec((_HEADS, _FIN_R, _HID), lambda i: (0, i, 0)),
            pl.BlockSpec((_HEADS, _FIN_R), lambda i: (0, i)),
            pl.BlockSpec((_HEADS, _HID), lambda i: (0, 0)),
            pl.BlockSpec((2 * _HID, _HID), lambda i: (0, 0)),
            pl.BlockSpec((1, _HID), lambda i: (0, 0)),
            pl.BlockSpec((1, 1), lambda i: (0, 0)),
            pl.BlockSpec((1, _HID), lambda i: (0, 0)),
            pl.BlockSpec((1, 1), lambda i: (0, 0)),
            pl.BlockSpec((1, 1, _FIN_R), lambda i: (i, 0, 0)),
        ],
        out_specs=[
            pl.BlockSpec((_FIN_R, 2 * _HID), lambda i: (i, 0)),
            pl.BlockSpec((_GRAPHS, 2 * _HID), lambda i: (0, 0)),
        ],
        out_shape=[
            jax.ShapeDtypeStruct((_NPAD, 2 * _HID), jnp.float32),
            jax.ShapeDtypeStruct((_GRAPHS, 2 * _HID), jnp.float32),
        ],
        scratch_shapes=[
            pltpu.VMEM((_GRAPHS, 2 * _HID), jnp.float32),
            pltpu.VMEM((_GRAPHS, _HID), jnp.float32),
        ],
    )(num2, den2, b2r, gw1, gb1r, agr, gw2r, gb2r, batch2d)


def kernel(x, edge_index, batch_idx, emb_table, W1, att_src1, att_dst1, b1,
           a1, W2, att_src2, att_dst2, b2, gw1, gb1, ag, gw2, gb2):
    f32 = jnp.float32
    xv = jnp.pad(x[:, 0], (0, _NPAD - _N))
    emb_pad = jnp.pad(emb_table, ((0, _VPAD - _VOCAB), (0, 0)))

    loop = jnp.arange(_N, dtype=edge_index.dtype)
    src = jnp.concatenate([edge_index[0], loop])
    dst = jnp.concatenate([edge_index[1], loop])
    npad = _EPAD - _ETOT
    src2d = jnp.pad(src, (0, npad)).reshape(_TILES, _CPT, _CHUNK)
    dst2d = jnp.pad(dst, (0, npad)).reshape(_TILES, _CPT, _CHUNK)
    z = jnp.zeros((_CHUNK, _HID), f32)
    batch2d = jnp.pad(batch_idx.astype(jnp.int32), (0, _NPAD - _N),
                      constant_values=_GRAPHS).reshape(
                          _NPAD // _FIN_R, 1, _FIN_R)

    _node_prep, _edges = _sc_kernels()
    t2, asv, adv = _prep(emb_pad, W1, att_src1, att_dst1)
    t2f = t2.reshape(_HEADS * _VPAD, _HID)
    hn1, asn1, adn1 = _node_prep(xv, t2f, asv.reshape(-1), adv.reshape(-1))
    num1, den1 = _edges(hn1, asn1, adn1, src2d, dst2d, z)

    h2, asn2, adn2 = _mid(num1.reshape(_HEADS, _NPAD, _HID),
                          den1.reshape(_HEADS, _NPAD),
                          b1.reshape(_HEADS, _HID),
                          jnp.asarray(a1, f32).reshape(1, 1),
                          W2, att_src2, att_dst2)
    num2, den2 = _edges(h2.reshape(_HEADS * _NPAD, _HID),
                        asn2.reshape(-1), adn2.reshape(-1),
                        src2d, dst2d, z)

    oc_pad, hidden = _final(num2.reshape(_HEADS, _NPAD, _HID),
                            den2.reshape(_HEADS, _NPAD),
                            b2.reshape(_HEADS, _HID),
                            gw1, gb1.reshape(1, _HID),
                            jnp.asarray(ag, f32).reshape(1, 1),
                            gw2.reshape(1, _HID),
                            jnp.asarray(gb2, f32).reshape(1, 1),
                            batch2d)
    return oc_pad[:_N], hidden


# speedup vs baseline: 1.1073x; 1.0910x over previous
import functools

import jax
import jax.numpy as jnp
from jax import lax
from jax.experimental import pallas as pl
from jax.experimental.pallas import tpu as pltpu
from jax.experimental.pallas import tpu_sc as plsc

_N = 10000
_NPAD = 10240
_VOCAB = 1001
_VPAD = 1024
_HEADS = 2
_HID = 128
_GRAPHS = 64
_ETOT = 320000 + _N
_CHUNK = 128
_TILES = 16
_CPT = 168
_G8 = _CPT // 8
_EPAD = _CPT * _CHUNK * _TILES
_NPT = _NPAD // _TILES
_NCH = _NPT // _CHUNK


def _prep_body(emb_ref, w_ref, as_ref, ad_ref, t2_ref, asv_ref, adv_ref):
    hh = jnp.dot(emb_ref[...], w_ref[...], preferred_element_type=jnp.float32)
    h0 = hh[:, :_HID]
    h1 = hh[:, _HID:]
    t2_ref[0] = h0
    t2_ref[1] = h1
    s0 = jnp.sum(h0 * as_ref[0][None, :], axis=1)
    s1 = jnp.sum(h1 * as_ref[1][None, :], axis=1)
    asv_ref[...] = jnp.concatenate([s0[None], s1[None]], axis=0)
    d0 = jnp.sum(h0 * ad_ref[0][None, :], axis=1)
    d1 = jnp.sum(h1 * ad_ref[1][None, :], axis=1)
    adv_ref[...] = jnp.concatenate([d0[None], d1[None]], axis=0)


def _prep(emb_pad, W1, att_s, att_d):
    return pl.pallas_call(
        _prep_body,
        out_shape=[
            jax.ShapeDtypeStruct((_HEADS, _VPAD, _HID), jnp.float32),
            jax.ShapeDtypeStruct((_HEADS, _VPAD), jnp.float32),
            jax.ShapeDtypeStruct((_HEADS, _VPAD), jnp.float32),
        ],
    )(emb_pad, W1, att_s, att_d)


def _node_prep_body(xv_hbm, t2_hbm, asv_hbm, adv_hbm, hn_hbm, asn_hbm,
                    adn_hbm, idxb, offb, rows, sb, sem):
    cid = lax.axis_index("c")
    sid = lax.axis_index("s")
    vbase = cid * _VPAD

    def chunk(j, carry):
        nb = sid * _NPT + j * _CHUNK
        pltpu.sync_copy(xv_hbm.at[pl.ds(nb, _CHUNK)], idxb)
        for k in range(_CHUNK // 16):
            sl = pl.ds(k * 16, 16)
            offb[sl] = idxb[sl] + vbase
        ob = cid * _NPAD + nb
        pltpu.async_copy(t2_hbm.at[offb], rows, sem).wait()
        pltpu.sync_copy(rows, hn_hbm.at[pl.ds(ob, _CHUNK)])
        pltpu.async_copy(asv_hbm.at[offb], sb, sem).wait()
        pltpu.sync_copy(sb, asn_hbm.at[pl.ds(ob, _CHUNK)])
        pltpu.async_copy(adv_hbm.at[offb], sb, sem).wait()
        pltpu.sync_copy(sb, adn_hbm.at[pl.ds(ob, _CHUNK)])
        return carry

    lax.fori_loop(0, _NCH, chunk, 0)


def _edges_body(hn_hbm, asn_hbm, adn_hbm, src_hbm, dst_hbm, z_hbm,
                num_hbm, den_hbm,
                src8, dst8, offs8, offd8, asb8, adb8, eeb8, rows_a, rows_b,
                num_sh, den_sh, sem_s, sem_g, sem_n):
    cid = lax.axis_index("c")
    sid = lax.axis_index("s")
    hbase = cid * _NPAD
    rbufs = (rows_a, rows_b)

    pltpu.sync_copy(z_hbm, rows_a)

    def zero_chunk(j, carry):
        nb = sid * _NPT + j * _CHUNK
        pltpu.sync_copy(rows_a, num_sh.at[pl.ds(nb, _CHUNK)])
        pltpu.sync_copy(rows_a.at[0], den_sh.at[pl.ds(nb, _CHUNK)])
        return carry

    lax.fori_loop(0, _NCH, zero_chunk, 0)
    plsc.subcore_barrier()

    lanes = lax.iota(jnp.int32, 16)

    def edge_group(g, carry):
        pltpu.sync_copy(src_hbm.at[sid, pl.ds(g * 8, 8)], src8)
        pltpu.sync_copy(dst_hbm.at[sid, pl.ds(g * 8, 8)], dst8)
        for c in range(8):
            for k in range(_CHUNK // 16):
                sl = pl.ds(k * 16, 16)
                offs8[c, sl] = src8[c, sl] + hbase
                offd8[c, sl] = dst8[c, sl] + hbase
        descs = []
        for c in range(8):
            descs.append(pltpu.async_copy(asn_hbm.at[offs8.at[c]],
                                          asb8.at[c], sem_s))
            descs.append(pltpu.async_copy(adn_hbm.at[offd8.at[c]],
                                          adb8.at[c], sem_s))
        early0 = pltpu.async_copy(hn_hbm.at[offs8.at[0]], rows_a, sem_g)
        early1 = pltpu.async_copy(hn_hbm.at[offs8.at[1]], rows_b, sem_g)
        for d in descs:
            d.wait()
        ebase0 = (sid * _CPT + g * 8) * _CHUNK
        for c in range(8):
            for k in range(_CHUNK // 16):
                sl = pl.ds(k * 16, 16)
                s = asb8[c, sl] + adb8[c, sl]
                lr = jnp.maximum(s, s * 0.2)
                valid = (ebase0 + c * _CHUNK + k * 16 + lanes) < _ETOT
                eeb8[c, sl] = jnp.exp(lr) * jnp.where(valid, 1.0, 0.0)
        dd = [pltpu.async_copy(eeb8.at[c], den_sh.at[dst8.at[c]], sem_s,
                               add=True) for c in range(8)]

        pend = {0: early0, 1: early1}
        scat = [None, None]
        for c in range(8):
            pend[c].wait()
            if 1 <= c < 7:
                if scat[(c + 1) % 2] is not None:
                    scat[(c + 1) % 2].wait()
                    scat[(c + 1) % 2] = None
                pend[c + 1] = pltpu.async_copy(hn_hbm.at[offs8.at[c + 1]],
                                               rbufs[(c + 1) % 2], sem_g)
            rows = rbufs[c % 2]

            def rowmul(q, rcarry):
                ee16 = eeb8[c, pl.ds(q * 16, 16)]
                for j in range(16):
                    b = jnp.full((16,), ee16[j], jnp.float32)
                    e = q * 16 + j
                    for k in range(_HID // 16):
                        sl = pl.ds(k * 16, 16)
                        rows[e, sl] = rows[e, sl] * b
                return rcarry

            lax.fori_loop(0, _CHUNK // 16, rowmul, 0)
            scat[c % 2] = pltpu.async_copy(rows, num_sh.at[dst8.at[c]],
                                           sem_n, add=True)
        for d in dd:
            d.wait()
        for s in scat:
            if s is not None:
                s.wait()
        return carry

    lax.fori_loop(0, _G8, edge_group, 0)
    plsc.subcore_barrier()

    def writeback(j, carry):
        nb = sid * _NPT + j * _CHUNK
        pltpu.sync_copy(num_sh.at[pl.ds(nb, _CHUNK)],
                        num_hbm.at[pl.ds(hbase + nb, _CHUNK)])
        pltpu.sync_copy(den_sh.at[pl.ds(nb, _CHUNK)],
                        den_hbm.at[pl.ds(hbase + nb, _CHUNK)])
        return carry

    lax.fori_loop(0, _NCH, writeback, 0)


@functools.cache
def _sc_kernels():
    mesh = plsc.VectorSubcoreMesh(core_axis_name="c", subcore_axis_name="s")
    node_prep = pl.kernel(
        _node_prep_body,
        out_type=[
            jax.ShapeDtypeStruct((_HEADS * _NPAD, _HID), jnp.float32),
            jax.ShapeDtypeStruct((_HEADS * _NPAD,), jnp.float32),
            jax.ShapeDtypeStruct((_HEADS * _NPAD,), jnp.float32),
        ],
        mesh=mesh,
        scratch_types=[
            pltpu.VMEM((_CHUNK,), jnp.int32),
            pltpu.VMEM((_CHUNK,), jnp.int32),
            pltpu.VMEM((_CHUNK, _HID), jnp.float32),
            pltpu.VMEM((_CHUNK,), jnp.float32),
            pltpu.SemaphoreType.DMA,
        ],
    )
    edges = pl.kernel(
        _edges_body,
        out_type=[
            jax.ShapeDtypeStruct((_HEADS * _NPAD, _HID), jnp.float32),
            jax.ShapeDtypeStruct((_HEADS * _NPAD,), jnp.float32),
        ],
        mesh=mesh,
        scratch_types=[
            pltpu.VMEM((8, _CHUNK), jnp.int32),
            pltpu.VMEM((8, _CHUNK), jnp.int32),
            pltpu.VMEM((8, _CHUNK), jnp.int32),
            pltpu.VMEM((8, _CHUNK), jnp.int32),
            pltpu.VMEM((8, _CHUNK), jnp.float32),
            pltpu.VMEM((8, _CHUNK), jnp.float32),
            pltpu.VMEM((8, _CHUNK), jnp.float32),
            pltpu.VMEM((_CHUNK, _HID), jnp.float32),
            pltpu.VMEM((_CHUNK, _HID), jnp.float32),
            pltpu.VMEM_SHARED((_NPAD, _HID), jnp.float32),
            pltpu.VMEM_SHARED((_NPAD,), jnp.float32),
            pltpu.SemaphoreType.DMA,
            pltpu.SemaphoreType.DMA,
            pltpu.SemaphoreType.DMA,
        ],
    )
    return node_prep, edges


def _mid_body(num_ref, den_ref, b1_ref, a1_ref, w2_ref, as2_ref, ad2_ref,
              h2_ref, asn2_ref, adn2_ref):
    o0 = num_ref[0] / (den_ref[0][:, None] + 1e-16) + b1_ref[0][None, :]
    o1 = num_ref[1] / (den_ref[1][:, None] + 1e-16) + b1_ref[1][None, :]
    x = jnp.concatenate([o0, o1], axis=1)
    a1 = a1_ref[0, 0]
    x = jnp.where(x >= 0, x, a1 * x)
    h2 = jnp.dot(x, w2_ref[...], preferred_element_type=jnp.float32)
    g0 = h2[:, :_HID]
    g1 = h2[:, _HID:]
    h2_ref[0] = g0
    h2_ref[1] = g1
    s0 = jnp.sum(g0 * as2_ref[0][None, :], axis=1)
    s1 = jnp.sum(g1 * as2_ref[1][None, :], axis=1)
    asn2_ref[...] = jnp.concatenate([s0[None], s1[None]], axis=0)
    d0 = jnp.sum(g0 * ad2_ref[0][None, :], axis=1)
    d1 = jnp.sum(g1 * ad2_ref[1][None, :], axis=1)
    adn2_ref[...] = jnp.concatenate([d0[None], d1[None]], axis=0)


_MID_R = 256


def _mid(num1, den1, b1r, a1r, W2, att_s2, att_d2):
    grid = (_NPAD // _MID_R,)
    return pl.pallas_call(
        _mid_body,
        grid=grid,
        in_specs=[
            pl.BlockSpec((_HEADS, _MID_R, _HID), lambda i: (0, i, 0)),
            pl.BlockSpec((_HEADS, _MID_R), lambda i: (0, i)),
            pl.BlockSpec((_HEADS, _HID), lambda i: (0, 0)),
            pl.BlockSpec((1, 1), lambda i: (0, 0)),
            pl.BlockSpec((2 * _HID, 2 * _HID), lambda i: (0, 0)),
            pl.BlockSpec((_HEADS, _HID), lambda i: (0, 0)),
            pl.BlockSpec((_HEADS, _HID), lambda i: (0, 0)),
        ],
        out_specs=[
            pl.BlockSpec((_HEADS, _MID_R, _HID), lambda i: (0, i, 0)),
            pl.BlockSpec((_HEADS, _MID_R), lambda i: (0, i)),
            pl.BlockSpec((_HEADS, _MID_R), lambda i: (0, i)),
        ],
        out_shape=[
            jax.ShapeDtypeStruct((_HEADS, _NPAD, _HID), jnp.float32),
            jax.ShapeDtypeStruct((_HEADS, _NPAD), jnp.float32),
            jax.ShapeDtypeStruct((_HEADS, _NPAD), jnp.float32),
        ],
    )(num1, den1, b1r, a1r, W2, att_s2, att_d2)


_FIN_R = 128


def _final_body(num_ref, den_ref, b2_ref, gw1_ref, gb1_ref, ag_ref, gw2_ref,
                gb2_ref, batch_ref, oc_ref, hid_ref, hid_acc, gden_acc):
    i = pl.program_id(0)
    o0 = num_ref[0] / (den_ref[0][:, None] + 1e-16) + b2_ref[0][None, :]
    o1 = num_ref[1] / (den_ref[1][:, None] + 1e-16) + b2_ref[1][None, :]
    oc = jnp.concatenate([o0, o1], axis=1)
    oc_ref[...] = oc
    g = jnp.dot(oc, gw1_ref[...], preferred_element_type=jnp.float32)
    g = g + gb1_ref[...]
    ag = ag_ref[0, 0]
    g = jnp.where(g >= 0, g, ag * g)
    gate = jnp.sum(g * gw2_ref[...], axis=1, keepdims=True) + gb2_ref[0, 0]
    ge = jnp.exp(gate)
    onehot = jnp.equal(
        lax.broadcasted_iota(jnp.int32, (_GRAPHS, _FIN_R), 0),
        batch_ref[0, 0][None, :]).astype(jnp.float32)

    @pl.when(i == 0)
    def _():
        hid_acc[...] = jnp.zeros_like(hid_acc)
        gden_acc[...] = jnp.zeros_like(gden_acc)

    hid_acc[...] += jnp.dot(onehot, oc * ge,
                            preferred_element_type=jnp.float32)
    gden_acc[...] += jnp.dot(onehot, jnp.broadcast_to(ge, (_FIN_R, _HID)),
                             preferred_element_type=jnp.float32)

    @pl.when(i == pl.num_programs(0) - 1)
    def _():
        hid_ref[...] = hid_acc[...] / (gden_acc[:, 0:1] + 1e-16)


def _final(num2, den2, b2r, gw1, gb1r, agr, gw2r, gb2r, batch2d):
    grid = (_NPAD // _FIN_R,)
    return pl.pallas_call(
        _final_body,
        grid=grid,
        in_specs=[
            pl.BlockSpec((_HEADS, _FIN_R, _HID), lambda i: (0, i, 0)),
            pl.BlockSpec((_HEADS, _FIN_R), lambda i: (0, i)),
            pl.BlockSpec((_HEADS, _HID), lambda i: (0, 0)),
            pl.BlockSpec((2 * _HID, _HID), lambda i: (0, 0)),
            pl.BlockSpec((1, _HID), lambda i: (0, 0)),
            pl.BlockSpec((1, 1), lambda i: (0, 0)),
            pl.BlockSpec((1, _HID), lambda i: (0, 0)),
            pl.BlockSpec((1, 1), lambda i: (0, 0)),
            pl.BlockSpec((1, 1, _FIN_R), lambda i: (i, 0, 0)),
        ],
        out_specs=[
            pl.BlockSpec((_FIN_R, 2 * _HID), lambda i: (i, 0)),
            pl.BlockSpec((_GRAPHS, 2 * _HID), lambda i: (0, 0)),
        ],
        out_shape=[
            jax.ShapeDtypeStruct((_NPAD, 2 * _HID), jnp.float32),
            jax.ShapeDtypeStruct((_GRAPHS, 2 * _HID), jnp.float32),
        ],
        scratch_shapes=[
            pltpu.VMEM((_GRAPHS, 2 * _HID), jnp.float32),
            pltpu.VMEM((_GRAPHS, _HID), jnp.float32),
        ],
    )(num2, den2, b2r, gw1, gb1r, agr, gw2r, gb2r, batch2d)


def kernel(x, edge_index, batch_idx, emb_table, W1, att_src1, att_dst1, b1,
           a1, W2, att_src2, att_dst2, b2, gw1, gb1, ag, gw2, gb2):
    f32 = jnp.float32
    xv = jnp.pad(x[:, 0], (0, _NPAD - _N))
    emb_pad = jnp.pad(emb_table, ((0, _VPAD - _VOCAB), (0, 0)))

    loop = jnp.arange(_N, dtype=edge_index.dtype)
    src = jnp.concatenate([edge_index[0], loop])
    dst = jnp.concatenate([edge_index[1], loop])
    npad = _EPAD - _ETOT
    src2d = jnp.pad(src, (0, npad)).reshape(_TILES, _CPT, _CHUNK)
    dst2d = jnp.pad(dst, (0, npad)).reshape(_TILES, _CPT, _CHUNK)
    z = jnp.zeros((_CHUNK, _HID), f32)
    batch2d = jnp.pad(batch_idx.astype(jnp.int32), (0, _NPAD - _N),
                      constant_values=_GRAPHS).reshape(
                          _NPAD // _FIN_R, 1, _FIN_R)

    _node_prep, _edges = _sc_kernels()
    t2, asv, adv = _prep(emb_pad, W1, att_src1, att_dst1)
    t2f = t2.reshape(_HEADS * _VPAD, _HID)
    hn1, asn1, adn1 = _node_prep(xv, t2f, asv.reshape(-1), adv.reshape(-1))
    num1, den1 = _edges(hn1, asn1, adn1, src2d, dst2d, z)

    h2, asn2, adn2 = _mid(num1.reshape(_HEADS, _NPAD, _HID),
                          den1.reshape(_HEADS, _NPAD),
                          b1.reshape(_HEADS, _HID),
                          jnp.asarray(a1, f32).reshape(1, 1),
                          W2, att_src2, att_dst2)
    num2, den2 = _edges(h2.reshape(_HEADS * _NPAD, _HID),
                        asn2.reshape(-1), adn2.reshape(-1),
                        src2d, dst2d, z)

    oc_pad, hidden = _final(num2.reshape(_HEADS, _NPAD, _HID),
                            den2.reshape(_HEADS, _NPAD),
                            b2.reshape(_HEADS, _HID),
                            gw1, gb1.reshape(1, _HID),
                            jnp.asarray(ag, f32).reshape(1, 1),
                            gw2.reshape(1, _HID),
                            jnp.asarray(gb2, f32).reshape(1, 1),
                            batch2d)
    return oc_pad[:_N], hidden
